# Initial kernel scaffold; baseline (speedup 1.0000x reference)
import jax
import jax.numpy as jnp
from jax.experimental import pallas as pl


def kernel(input, rois):
    raise NotImplementedError("write your pallas kernel here")


# speedup vs baseline: 10.1398x; 10.1398x over previous
import functools

import jax
import jax.numpy as jnp
from jax import lax
from jax.experimental import pallas as pl
from jax.experimental.pallas import tpu as pltpu
from jax.experimental.pallas import tpu_sc as plsc

OUT_H = 7
OUT_W = 7
SPATIAL_SCALE = 0.125
SAMPLING_RATIO = 2
TAPS = SAMPLING_RATIO * SAMPLING_RATIO * 4
LANES = 16


def _roi_index_kernel(shapes, rois_t_ref, idx_ref, wgt_ref):
    N, H, W = shapes
    n = rois_t_ref.shape[1]
    gh = gw = SAMPLING_RATIO
    ns = OUT_H * OUT_W * gh * gw

    b = rois_t_ref[0].astype(jnp.int32)
    cw = rois_t_ref[1] * SPATIAL_SCALE - 0.5
    ch = rois_t_ref[2] * SPATIAL_SCALE - 0.5
    rw = rois_t_ref[3] * SPATIAL_SCALE
    rh = rois_t_ref[4] * SPATIAL_SCALE
    theta = rois_t_ref[5]
    cosT = jnp.cos(theta)[:, None]
    sinT = jnp.sin(theta)[:, None]

    s = lax.broadcasted_iota(jnp.int32, (n, ns), 1)
    ph = (s // (OUT_W * gh * gw)).astype(jnp.float32)
    pw = ((s // (gh * gw)) % OUT_W).astype(jnp.float32)
    iy = ((s % (gh * gw)) // gw).astype(jnp.float32)
    ix = (s % gw).astype(jnp.float32)

    bin_h = (rh / OUT_H)[:, None]
    bin_w = (rw / OUT_W)[:, None]
    yy = (-rh / 2.0)[:, None] + ph * bin_h + (iy + 0.5) * bin_h / gh
    xx = (-rw / 2.0)[:, None] + pw * bin_w + (ix + 0.5) * bin_w / gw
    y = yy * cosT - xx * sinT + ch[:, None]
    x = yy * sinT + xx * cosT + cw[:, None]

    valid = (y >= -1.0) & (y <= float(H)) & (x >= -1.0) & (x <= float(W))
    y = jnp.maximum(y, 0.0)
    x = jnp.maximum(x, 0.0)
    yl0 = jnp.floor(y).astype(jnp.int32)
    xl0 = jnp.floor(x).astype(jnp.int32)
    ycond = yl0 >= H - 1
    xcond = xl0 >= W - 1
    y_low = jnp.where(ycond, H - 1, yl0)
    y_high = jnp.where(ycond, H - 1, yl0 + 1)
    y = jnp.where(ycond, float(H - 1), y)
    x_low = jnp.where(xcond, W - 1, xl0)
    x_high = jnp.where(xcond, W - 1, xl0 + 1)
    x = jnp.where(xcond, float(W - 1), x)
    ly = y - y_low.astype(jnp.float32)
    lx = x - x_low.astype(jnp.float32)
    hy = 1.0 - ly
    hx = 1.0 - lx
    vm = valid.astype(jnp.float32) * (1.0 / (gh * gw))

    base = b[:, None] * (H * W)
    idx_ref[0] = base + y_low * W + x_low
    idx_ref[1] = base + y_low * W + x_high
    idx_ref[2] = base + y_high * W + x_low
    idx_ref[3] = base + y_high * W + x_high
    wgt_ref[0] = hy * hx * vm
    wgt_ref[1] = hy * lx * vm
    wgt_ref[2] = ly * hx * vm
    wgt_ref[3] = ly * lx * vm


def _bcast_lane(v, j):
    dn = lax.GatherDimensionNumbers(
        offset_dims=(), collapsed_slice_dims=(0,), start_index_map=(0,)
    )
    return lax.gather(
        v,
        jnp.full((LANES, 1), j, jnp.int32),
        dn,
        slice_sizes=(1,),
        mode=lax.GatherScatterMode.PROMISE_IN_BOUNDS,
    )


def _make_sc_bag(n_rows_pad, C, n_workers, num_cores):
    rows_per_w = n_rows_pad // n_workers
    CHUNK = 8
    n_chunks = rows_per_w // CHUNK
    cchunks = C // LANES
    mesh = plsc.VectorSubcoreMesh(core_axis_name="c", subcore_axis_name="s")

    @functools.partial(
        pl.kernel,
        mesh=mesh,
        out_type=jax.ShapeDtypeStruct((n_rows_pad, C), jnp.float32),
        scratch_types=[
            pltpu.VMEM((CHUNK * TAPS,), jnp.int32),
            pltpu.VMEM((CHUNK * TAPS,), jnp.int32),
            pltpu.VMEM((CHUNK * TAPS,), jnp.float32),
            pltpu.VMEM((CHUNK * TAPS,), jnp.float32),
            pltpu.VMEM((CHUNK * TAPS, C), jnp.float32),
            pltpu.VMEM((CHUNK * TAPS, C), jnp.float32),
            pltpu.VMEM((CHUNK, C), jnp.float32),
            pltpu.SemaphoreType.DMA,
            pltpu.SemaphoreType.DMA,
        ],
    )
    def bag(table, idxf, wgtf, out, idx0, idx1, w0, w1, r0, r1, accv, s0, s1):
        idxb = (idx0, idx1)
        wgtb = (w0, w1)
        rowb = (r0, r1)
        semb = (s0, s1)
        wid = lax.axis_index("s") * num_cores + lax.axis_index("c")
        row0 = wid * rows_per_w

        def start(g, b):
            off = (row0 + g * CHUNK) * TAPS
            pltpu.sync_copy(idxf.at[pl.ds(off, CHUNK * TAPS)], idxb[b])
            pltpu.sync_copy(wgtf.at[pl.ds(off, CHUNK * TAPS)], wgtb[b])
            pltpu.async_copy(table.at[idxb[b]], rowb[b], semb[b])

        start(0, 0)
        start(1, 1)

        def outer(i, carry):
            for b in (0, 1):
                g = i * 2 + b
                pltpu.make_async_copy(table.at[idxb[b]], rowb[b], semb[b]).wait()

                def row_body(r, c2, _rows=rowb[b], _wg=wgtb[b]):
                    w16 = _wg[pl.ds(r * TAPS, TAPS)]
                    wjs = [_bcast_lane(w16, j) for j in range(TAPS)]
                    for cc in range(cchunks):
                        acc = wjs[0] * _rows[r * TAPS, pl.ds(cc * LANES, LANES)]
                        for j in range(1, TAPS):
                            acc = acc + wjs[j] * _rows[
                                r * TAPS + j, pl.ds(cc * LANES, LANES)
                            ]
                        accv[r, pl.ds(cc * LANES, LANES)] = acc
                    return c2

                lax.fori_loop(0, CHUNK, row_body, 0)
                pltpu.sync_copy(accv, out.at[pl.ds(row0 + g * CHUNK, CHUNK)])
                g2 = jnp.minimum(g + 2, n_chunks - 1)
                start(g2, b)
            return carry

        lax.fori_loop(0, n_chunks // 2, outer, 0)
        for b in (0, 1):
            pltpu.make_async_copy(table.at[idxb[b]], rowb[b], semb[b]).wait()

    return bag


def kernel(input, rois):
    N, C, H, W = input.shape
    n = rois.shape[0]
    gh = gw = SAMPLING_RATIO
    n_bins = OUT_H * OUT_W

    info = plsc.get_sparse_core_info()
    n_workers = info.num_cores * info.num_subcores

    table = jnp.transpose(input, (0, 2, 3, 1)).reshape(N * H * W, C)

    idx4, wgt4 = pl.pallas_call(
        functools.partial(_roi_index_kernel, (N, H, W)),
        out_shape=[
            jax.ShapeDtypeStruct((4, n, n_bins * gh * gw), jnp.int32),
            jax.ShapeDtypeStruct((4, n, n_bins * gh * gw), jnp.float32),
        ],
    )(rois.T)

    idx4 = idx4.reshape(4, n, n_bins, gh * gw)
    wgt4 = wgt4.reshape(4, n, n_bins, gh * gw)
    idxf = jnp.transpose(idx4, (1, 2, 3, 0)).reshape(n * n_bins * TAPS)
    wgtf = jnp.transpose(wgt4, (1, 2, 3, 0)).reshape(n * n_bins * TAPS)

    n_rows = n * n_bins
    quant = n_workers * 8
    n_rows_pad = ((n_rows + quant - 1) // quant) * quant
    pad = (n_rows_pad - n_rows) * TAPS
    if pad:
        idxf = jnp.concatenate([idxf, jnp.zeros((pad,), jnp.int32)])
        wgtf = jnp.concatenate([wgtf, jnp.zeros((pad,), jnp.float32)])

    bag = _make_sc_bag(n_rows_pad, C, n_workers, info.num_cores)
    out = bag(table, idxf, wgtf)

    out = out[:n_rows].reshape(n, n_bins, C)
    out = jnp.transpose(out, (0, 2, 1)).reshape(n, C, OUT_H, OUT_W)
    return out


# speedup vs baseline: 12.9469x; 1.2768x over previous
import functools

import jax
import jax.numpy as jnp
from jax import lax
from jax.experimental import pallas as pl
from jax.experimental.pallas import tpu as pltpu
from jax.experimental.pallas import tpu_sc as plsc

OUT_H = 7
OUT_W = 7
SPATIAL_SCALE = 0.125
SAMPLING_RATIO = 2
TAPS = SAMPLING_RATIO * SAMPLING_RATIO * 4
LANES = 16


def _roi_index_kernel(shapes, rois_t_ref, idx_ref, wgt_ref):
    N, H, W, n_real = shapes
    nb = rois_t_ref.shape[1]
    gh = gw = SAMPLING_RATIO
    ns = OUT_H * OUT_W * gh * gw * 4

    b = rois_t_ref[0].astype(jnp.int32)
    cw = rois_t_ref[1] * SPATIAL_SCALE - 0.5
    ch = rois_t_ref[2] * SPATIAL_SCALE - 0.5
    rw = rois_t_ref[3] * SPATIAL_SCALE
    rh = rois_t_ref[4] * SPATIAL_SCALE
    theta = rois_t_ref[5]
    cosT = jnp.cos(theta)[:, None]
    sinT = jnp.sin(theta)[:, None]

    s = lax.broadcasted_iota(jnp.int32, (nb, ns), 1)
    c4 = s % 4
    t = s // 4
    ph = (t // (4 * OUT_W)).astype(jnp.float32)
    pw = ((t // 4) % OUT_W).astype(jnp.float32)
    iy = ((t % 4) // 2).astype(jnp.float32)
    ix = (t % 2).astype(jnp.float32)

    bin_h = (rh / OUT_H)[:, None]
    bin_w = (rw / OUT_W)[:, None]
    yy = (-rh / 2.0)[:, None] + ph * bin_h + (iy + 0.5) * bin_h / gh
    xx = (-rw / 2.0)[:, None] + pw * bin_w + (ix + 0.5) * bin_w / gw
    y = yy * cosT - xx * sinT + ch[:, None]
    x = yy * sinT + xx * cosT + cw[:, None]

    valid = (y >= -1.0) & (y <= float(H)) & (x >= -1.0) & (x <= float(W))
    y = jnp.maximum(y, 0.0)
    x = jnp.maximum(x, 0.0)
    yl0 = jnp.floor(y).astype(jnp.int32)
    xl0 = jnp.floor(x).astype(jnp.int32)
    ycond = yl0 >= H - 1
    xcond = xl0 >= W - 1
    y_low = jnp.where(ycond, H - 1, yl0)
    y_high = jnp.where(ycond, H - 1, yl0 + 1)
    y = jnp.where(ycond, float(H - 1), y)
    x_low = jnp.where(xcond, W - 1, xl0)
    x_high = jnp.where(xcond, W - 1, xl0 + 1)
    x = jnp.where(xcond, float(W - 1), x)
    ly = y - y_low.astype(jnp.float32)
    lx = x - x_low.astype(jnp.float32)
    hy = 1.0 - ly
    hx = 1.0 - lx
    vm = valid.astype(jnp.float32) * (1.0 / (gh * gw))
    row = pl.program_id(0) * nb + lax.broadcasted_iota(jnp.int32, (nb, ns), 0)
    vm = jnp.where(row < n_real, vm, 0.0)

    hi_y = c4 >= 2
    hi_x = (c4 % 2) == 1
    y_sel = jnp.where(hi_y, y_high, y_low)
    x_sel = jnp.where(hi_x, x_high, x_low)
    wy = jnp.where(hi_y, ly, hy)
    wx = jnp.where(hi_x, lx, hx)
    idx_ref[...] = b[:, None] * (H * W) + y_sel * W + x_sel
    wgt_ref[...] = wy * wx * vm


def _transpose_kernel(in_ref, out_ref):
    out_ref[...] = in_ref[0].T


def _bcast_lane(v, j):
    dn = lax.GatherDimensionNumbers(
        offset_dims=(), collapsed_slice_dims=(0,), start_index_map=(0,)
    )
    return lax.gather(
        v,
        jnp.full((LANES, 1), j, jnp.int32),
        dn,
        slice_sizes=(1,),
        mode=lax.GatherScatterMode.PROMISE_IN_BOUNDS,
    )


def _make_sc_bag(n_rows_pad, C, n_workers, num_cores):
    rows_per_w = n_rows_pad // n_workers
    CHUNK = 8
    n_chunks = rows_per_w // CHUNK
    cchunks = C // LANES
    mesh = plsc.VectorSubcoreMesh(core_axis_name="c", subcore_axis_name="s")

    @functools.partial(
        pl.kernel,
        mesh=mesh,
        out_type=jax.ShapeDtypeStruct((n_rows_pad, C), jnp.float32),
        scratch_types=[
            pltpu.VMEM((CHUNK * TAPS,), jnp.int32),
            pltpu.VMEM((CHUNK * TAPS,), jnp.int32),
            pltpu.VMEM((CHUNK * TAPS,), jnp.float32),
            pltpu.VMEM((CHUNK * TAPS,), jnp.float32),
            pltpu.VMEM((CHUNK * TAPS, C), jnp.float32),
            pltpu.VMEM((CHUNK * TAPS, C), jnp.float32),
            pltpu.VMEM((CHUNK, C), jnp.float32),
            pltpu.SemaphoreType.DMA,
            pltpu.SemaphoreType.DMA,
        ],
    )
    def bag(table, idxf, wgtf, out, idx0, idx1, w0, w1, r0, r1, accv, s0, s1):
        idxb = (idx0, idx1)
        wgtb = (w0, w1)
        rowb = (r0, r1)
        semb = (s0, s1)
        wid = lax.axis_index("s") * num_cores + lax.axis_index("c")
        row0 = wid * rows_per_w

        def start(g, b):
            off = (row0 + g * CHUNK) * TAPS
            pltpu.sync_copy(idxf.at[pl.ds(off, CHUNK * TAPS)], idxb[b])
            pltpu.sync_copy(wgtf.at[pl.ds(off, CHUNK * TAPS)], wgtb[b])
            pltpu.async_copy(table.at[idxb[b]], rowb[b], semb[b])

        start(0, 0)
        start(1, 1)

        def outer(i, carry):
            for b in (0, 1):
                g = i * 2 + b
                pltpu.make_async_copy(table.at[idxb[b]], rowb[b], semb[b]).wait()

                def row_body(r, c2, _rows=rowb[b], _wg=wgtb[b]):
                    w16 = _wg[pl.ds(r * TAPS, TAPS)]
                    wjs = [_bcast_lane(w16, j) for j in range(TAPS)]
                    for cc in range(cchunks):
                        acc = wjs[0] * _rows[r * TAPS, pl.ds(cc * LANES, LANES)]
                        for j in range(1, TAPS):
                            acc = acc + wjs[j] * _rows[
                                r * TAPS + j, pl.ds(cc * LANES, LANES)
                            ]
                        accv[r, pl.ds(cc * LANES, LANES)] = acc
                    return c2

                lax.fori_loop(0, CHUNK, row_body, 0)
                pltpu.sync_copy(accv, out.at[pl.ds(row0 + g * CHUNK, CHUNK)])
                g2 = jnp.minimum(g + 2, n_chunks - 1)
                start(g2, b)
            return carry

        lax.fori_loop(0, n_chunks // 2, outer, 0)
        for b in (0, 1):
            pltpu.make_async_copy(table.at[idxb[b]], rowb[b], semb[b]).wait()

    return bag


def kernel(input, rois):
    N, C, H, W = input.shape
    n = rois.shape[0]
    n_bins = OUT_H * OUT_W

    info = plsc.get_sparse_core_info()
    n_workers = info.num_cores * info.num_subcores
    n_pad = n
    while (n_pad * n_bins) % (n_workers * 8):
        n_pad += 8
    n_rows_pad = n_pad * n_bins

    HWC = H * W
    hw_blk = 2048
    table = pl.pallas_call(
        _transpose_kernel,
        grid=(N, HWC // hw_blk),
        in_specs=[
            pl.BlockSpec((1, C, hw_blk), lambda b, j: (b, 0, j)),
        ],
        out_specs=pl.BlockSpec(
            (hw_blk, C), lambda b, j: (b * (HWC // hw_blk) + j, 0)
        ),
        out_shape=jax.ShapeDtypeStruct((N * HWC, C), jnp.float32),
    )(input.reshape(N, C, HWC))

    rois_t = rois.T
    if n_pad != n:
        rois_t = jnp.pad(rois_t, ((0, 0), (0, n_pad - n)))
    roi_blk = 128
    ns = n_bins * TAPS
    idx2, wgt2 = pl.pallas_call(
        functools.partial(_roi_index_kernel, (N, H, W, n)),
        grid=(n_pad // roi_blk,),
        in_specs=[pl.BlockSpec((6, roi_blk), lambda i: (0, i))],
        out_specs=[
            pl.BlockSpec((roi_blk, ns), lambda i: (i, 0)),
            pl.BlockSpec((roi_blk, ns), lambda i: (i, 0)),
        ],
        out_shape=[
            jax.ShapeDtypeStruct((n_pad, ns), jnp.int32),
            jax.ShapeDtypeStruct((n_pad, ns), jnp.float32),
        ],
    )(rois_t)
    idxf = idx2.reshape(n_pad * ns)
    wgtf = wgt2.reshape(n_pad * ns)

    bag = _make_sc_bag(n_rows_pad, C, n_workers, info.num_cores)
    out = bag(table, idxf, wgtf)

    out = out[: n * n_bins].reshape(n, n_bins, C)
    out = jnp.transpose(out, (0, 2, 1)).reshape(n, C, OUT_H, OUT_W)
    return out


# speedup vs baseline: 13.9196x; 1.0751x over previous
import functools

import jax
import jax.numpy as jnp
from jax import lax
from jax.experimental import pallas as pl
from jax.experimental.pallas import tpu as pltpu
from jax.experimental.pallas import tpu_sc as plsc

OUT_H = 7
OUT_W = 7
SPATIAL_SCALE = 0.125
SAMPLING_RATIO = 2
TAPS = SAMPLING_RATIO * SAMPLING_RATIO * 4
LANES = 16


def _roi_index_kernel(shapes, rois_t_ref, idx_ref, wgt_ref):
    N, H, W, n_real = shapes
    nb = rois_t_ref.shape[1]
    gh = gw = SAMPLING_RATIO
    ns = OUT_H * OUT_W * gh * gw * 4

    b = rois_t_ref[0].astype(jnp.int32)
    cw = rois_t_ref[1] * SPATIAL_SCALE - 0.5
    ch = rois_t_ref[2] * SPATIAL_SCALE - 0.5
    rw = rois_t_ref[3] * SPATIAL_SCALE
    rh = rois_t_ref[4] * SPATIAL_SCALE
    theta = rois_t_ref[5]
    cosT = jnp.cos(theta)[:, None]
    sinT = jnp.sin(theta)[:, None]

    s = lax.broadcasted_iota(jnp.int32, (nb, ns), 1)
    c4 = s % 4
    t = s // 4
    ph = (t // (4 * OUT_W)).astype(jnp.float32)
    pw = ((t // 4) % OUT_W).astype(jnp.float32)
    iy = ((t % 4) // 2).astype(jnp.float32)
    ix = (t % 2).astype(jnp.float32)

    bin_h = (rh / OUT_H)[:, None]
    bin_w = (rw / OUT_W)[:, None]
    yy = (-rh / 2.0)[:, None] + ph * bin_h + (iy + 0.5) * bin_h / gh
    xx = (-rw / 2.0)[:, None] + pw * bin_w + (ix + 0.5) * bin_w / gw
    y = yy * cosT - xx * sinT + ch[:, None]
    x = yy * sinT + xx * cosT + cw[:, None]

    valid = (y >= -1.0) & (y <= float(H)) & (x >= -1.0) & (x <= float(W))
    y = jnp.maximum(y, 0.0)
    x = jnp.maximum(x, 0.0)
    yl0 = jnp.floor(y).astype(jnp.int32)
    xl0 = jnp.floor(x).astype(jnp.int32)
    ycond = yl0 >= H - 1
    xcond = xl0 >= W - 1
    y_low = jnp.where(ycond, H - 1, yl0)
    y_high = jnp.where(ycond, H - 1, yl0 + 1)
    y = jnp.where(ycond, float(H - 1), y)
    x_low = jnp.where(xcond, W - 1, xl0)
    x_high = jnp.where(xcond, W - 1, xl0 + 1)
    x = jnp.where(xcond, float(W - 1), x)
    ly = y - y_low.astype(jnp.float32)
    lx = x - x_low.astype(jnp.float32)
    hy = 1.0 - ly
    hx = 1.0 - lx
    vm = valid.astype(jnp.float32) * (1.0 / (gh * gw))
    row = pl.program_id(0) * nb + lax.broadcasted_iota(jnp.int32, (nb, ns), 0)
    vm = jnp.where(row < n_real, vm, 0.0)

    hi_y = c4 >= 2
    hi_x = (c4 % 2) == 1
    y_sel = jnp.where(hi_y, y_high, y_low)
    x_sel = jnp.where(hi_x, x_high, x_low)
    wy = jnp.where(hi_y, ly, hy)
    wx = jnp.where(hi_x, lx, hx)
    idx_ref[...] = b[:, None] * (H * W) + y_sel * W + x_sel
    wgt_ref[...] = wy * wx * vm


def _transpose_kernel(in_ref, out_ref):
    out_ref[...] = in_ref[0].T


def _out_relayout_kernel(in_ref, out_ref):
    out_ref[...] = jnp.transpose(in_ref[...], (0, 2, 1))


def _bcast_lane(v, j):
    dn = lax.GatherDimensionNumbers(
        offset_dims=(), collapsed_slice_dims=(0,), start_index_map=(0,)
    )
    return lax.gather(
        v,
        jnp.full((LANES, 1), j, jnp.int32),
        dn,
        slice_sizes=(1,),
        mode=lax.GatherScatterMode.PROMISE_IN_BOUNDS,
    )


def _make_sc_bag(n_rows_pad, C, n_workers, num_cores):
    rows_per_w = n_rows_pad // n_workers
    CHUNK = 8
    n_chunks = rows_per_w // CHUNK
    cchunks = C // LANES
    mesh = plsc.VectorSubcoreMesh(core_axis_name="c", subcore_axis_name="s")

    @functools.partial(
        pl.kernel,
        mesh=mesh,
        out_type=jax.ShapeDtypeStruct((n_rows_pad, C), jnp.float32),
        scratch_types=[
            pltpu.VMEM((CHUNK * TAPS,), jnp.int32),
            pltpu.VMEM((CHUNK * TAPS,), jnp.int32),
            pltpu.VMEM((CHUNK * TAPS,), jnp.float32),
            pltpu.VMEM((CHUNK * TAPS,), jnp.float32),
            pltpu.VMEM((CHUNK * TAPS, C), jnp.float32),
            pltpu.VMEM((CHUNK * TAPS, C), jnp.float32),
            pltpu.VMEM((CHUNK, C), jnp.float32),
            pltpu.SemaphoreType.DMA,
            pltpu.SemaphoreType.DMA,
        ],
    )
    def bag(table, idxf, wgtf, out, idx0, idx1, w0, w1, r0, r1, accv, s0, s1):
        idxb = (idx0, idx1)
        wgtb = (w0, w1)
        rowb = (r0, r1)
        semb = (s0, s1)
        wid = lax.axis_index("s") * num_cores + lax.axis_index("c")
        row0 = wid * rows_per_w

        def start(g, b):
            off = (row0 + g * CHUNK) * TAPS
            pltpu.sync_copy(idxf.at[pl.ds(off, CHUNK * TAPS)], idxb[b])
            pltpu.sync_copy(wgtf.at[pl.ds(off, CHUNK * TAPS)], wgtb[b])
            pltpu.async_copy(table.at[idxb[b]], rowb[b], semb[b])

        start(0, 0)
        start(1, 1)

        def outer(i, carry):
            for b in (0, 1):
                g = i * 2 + b
                pltpu.make_async_copy(table.at[idxb[b]], rowb[b], semb[b]).wait()

                def row_body(r, c2, _rows=rowb[b], _wg=wgtb[b]):
                    w16 = _wg[pl.ds(r * TAPS, TAPS)]
                    wjs = [_bcast_lane(w16, j) for j in range(TAPS)]
                    for cc in range(cchunks):
                        parts = [None] * 4
                        for j in range(TAPS):
                            v = _rows[r * TAPS + j, pl.ds(cc * LANES, LANES)]
                            k = j % 4
                            parts[k] = (
                                wjs[j] * v
                                if parts[k] is None
                                else parts[k] + wjs[j] * v
                            )
                        accv[r, pl.ds(cc * LANES, LANES)] = (
                            parts[0] + parts[1]
                        ) + (parts[2] + parts[3])
                    return c2

                lax.fori_loop(0, CHUNK, row_body, 0)
                pltpu.sync_copy(accv, out.at[pl.ds(row0 + g * CHUNK, CHUNK)])
                g2 = jnp.minimum(g + 2, n_chunks - 1)
                start(g2, b)
            return carry

        lax.fori_loop(0, n_chunks // 2, outer, 0)
        for b in (0, 1):
            pltpu.make_async_copy(table.at[idxb[b]], rowb[b], semb[b]).wait()

    return bag


def kernel(input, rois):
    N, C, H, W = input.shape
    n = rois.shape[0]
    n_bins = OUT_H * OUT_W

    info = plsc.get_sparse_core_info()
    n_workers = info.num_cores * info.num_subcores
    n_pad = n
    while (n_pad * n_bins) % (n_workers * 8):
        n_pad += 8
    n_rows_pad = n_pad * n_bins

    HWC = H * W
    hw_blk = 2048
    table = pl.pallas_call(
        _transpose_kernel,
        grid=(N, HWC // hw_blk),
        in_specs=[
            pl.BlockSpec((1, C, hw_blk), lambda b, j: (b, 0, j)),
        ],
        out_specs=pl.BlockSpec(
            (hw_blk, C), lambda b, j: (b * (HWC // hw_blk) + j, 0)
        ),
        out_shape=jax.ShapeDtypeStruct((N * HWC, C), jnp.float32),
    )(input.reshape(N, C, HWC))

    rois_t = rois.T
    if n_pad != n:
        rois_t = jnp.pad(rois_t, ((0, 0), (0, n_pad - n)))
    roi_blk = 128
    ns = n_bins * TAPS
    idx2, wgt2 = pl.pallas_call(
        functools.partial(_roi_index_kernel, (N, H, W, n)),
        grid=(n_pad // roi_blk,),
        in_specs=[pl.BlockSpec((6, roi_blk), lambda i: (0, i))],
        out_specs=[
            pl.BlockSpec((roi_blk, ns), lambda i: (i, 0)),
            pl.BlockSpec((roi_blk, ns), lambda i: (i, 0)),
        ],
        out_shape=[
            jax.ShapeDtypeStruct((n_pad, ns), jnp.int32),
            jax.ShapeDtypeStruct((n_pad, ns), jnp.float32),
        ],
    )(rois_t)
    idxf = idx2.reshape(n_pad * ns)
    wgtf = wgt2.reshape(n_pad * ns)

    bag = _make_sc_bag(n_rows_pad, C, n_workers, info.num_cores)
    out = bag(table, idxf, wgtf)

    roi_oblk = 8
    out_t = pl.pallas_call(
        _out_relayout_kernel,
        grid=(n_pad // roi_oblk,),
        in_specs=[pl.BlockSpec((roi_oblk, n_bins, C), lambda i: (i, 0, 0))],
        out_specs=pl.BlockSpec((roi_oblk, C, n_bins), lambda i: (i, 0, 0)),
        out_shape=jax.ShapeDtypeStruct((n_pad, C, n_bins), jnp.float32),
    )(out.reshape(n_pad, n_bins, C))
    return out_t[:n].reshape(n, C, OUT_H, OUT_W)


# speedup vs baseline: 15.2370x; 1.0946x over previous
import functools

import jax
import jax.numpy as jnp
from jax import lax
from jax.experimental import pallas as pl
from jax.experimental.pallas import tpu as pltpu
from jax.experimental.pallas import tpu_sc as plsc

OUT_H = 7
OUT_W = 7
SPATIAL_SCALE = 0.125
SAMPLING_RATIO = 2
TAPS = SAMPLING_RATIO * SAMPLING_RATIO * 4
LANES = 16


def _roi_index_kernel(shapes, rois_t_ref, idx_ref, wgt_ref):
    N, H, W, n_real = shapes
    nb = rois_t_ref.shape[1]
    gh = gw = SAMPLING_RATIO
    ns = OUT_H * OUT_W * gh * gw * 4

    b = rois_t_ref[0].astype(jnp.int32)
    cw = rois_t_ref[1] * SPATIAL_SCALE - 0.5
    ch = rois_t_ref[2] * SPATIAL_SCALE - 0.5
    rw = rois_t_ref[3] * SPATIAL_SCALE
    rh = rois_t_ref[4] * SPATIAL_SCALE
    theta = rois_t_ref[5]
    cosT = jnp.cos(theta)[:, None]
    sinT = jnp.sin(theta)[:, None]

    s = lax.broadcasted_iota(jnp.int32, (nb, ns), 1)
    c4 = s % 4
    t = s // 4
    ph = (t // (4 * OUT_W)).astype(jnp.float32)
    pw = ((t // 4) % OUT_W).astype(jnp.float32)
    iy = ((t % 4) // 2).astype(jnp.float32)
    ix = (t % 2).astype(jnp.float32)

    bin_h = (rh / OUT_H)[:, None]
    bin_w = (rw / OUT_W)[:, None]
    yy = (-rh / 2.0)[:, None] + ph * bin_h + (iy + 0.5) * bin_h / gh
    xx = (-rw / 2.0)[:, None] + pw * bin_w + (ix + 0.5) * bin_w / gw
    y = yy * cosT - xx * sinT + ch[:, None]
    x = yy * sinT + xx * cosT + cw[:, None]

    valid = (y >= -1.0) & (y <= float(H)) & (x >= -1.0) & (x <= float(W))
    y = jnp.maximum(y, 0.0)
    x = jnp.maximum(x, 0.0)
    yl0 = jnp.floor(y).astype(jnp.int32)
    xl0 = jnp.floor(x).astype(jnp.int32)
    ycond = yl0 >= H - 1
    xcond = xl0 >= W - 1
    y_low = jnp.where(ycond, H - 1, yl0)
    y_high = jnp.where(ycond, H - 1, yl0 + 1)
    y = jnp.where(ycond, float(H - 1), y)
    x_low = jnp.where(xcond, W - 1, xl0)
    x_high = jnp.where(xcond, W - 1, xl0 + 1)
    x = jnp.where(xcond, float(W - 1), x)
    ly = y - y_low.astype(jnp.float32)
    lx = x - x_low.astype(jnp.float32)
    hy = 1.0 - ly
    hx = 1.0 - lx
    vm = valid.astype(jnp.float32) * (1.0 / (gh * gw))
    row = pl.program_id(0) * nb + lax.broadcasted_iota(jnp.int32, (nb, ns), 0)
    vm = jnp.where(row < n_real, vm, 0.0)

    hi_y = c4 >= 2
    hi_x = (c4 % 2) == 1
    y_sel = jnp.where(hi_y, y_high, y_low)
    x_sel = jnp.where(hi_x, x_high, x_low)
    wy = jnp.where(hi_y, ly, hy)
    wx = jnp.where(hi_x, lx, hx)
    idx_ref[...] = b[:, None] * (H * W) + y_sel * W + x_sel
    wgt_ref[...] = wy * wx * vm


def _transpose_kernel(in_ref, out_ref):
    out_ref[...] = in_ref[0].T


def _out_relayout_kernel(in_ref, out_ref):
    out_ref[...] = jnp.transpose(in_ref[...], (0, 2, 1))


def _bcast_lane(v, j):
    dn = lax.GatherDimensionNumbers(
        offset_dims=(), collapsed_slice_dims=(0,), start_index_map=(0,)
    )
    return lax.gather(
        v,
        jnp.full((LANES, 1), j, jnp.int32),
        dn,
        slice_sizes=(1,),
        mode=lax.GatherScatterMode.PROMISE_IN_BOUNDS,
    )


def _make_sc_bag(n_rows_pad, C, n_workers, num_cores):
    rows_per_w = n_rows_pad // n_workers
    CHUNK = 8
    n_chunks = rows_per_w // CHUNK
    cchunks = C // LANES
    mesh = plsc.VectorSubcoreMesh(core_axis_name="c", subcore_axis_name="s")

    @functools.partial(
        pl.kernel,
        mesh=mesh,
        out_type=jax.ShapeDtypeStruct((n_rows_pad, C), jnp.float32),
        scratch_types=[
            pltpu.VMEM((rows_per_w * TAPS,), jnp.int32),
            pltpu.VMEM((rows_per_w * TAPS,), jnp.float32),
            pltpu.VMEM((CHUNK * TAPS, C), jnp.float32),
            pltpu.VMEM((CHUNK * TAPS, C), jnp.float32),
            pltpu.VMEM((CHUNK, C), jnp.float32),
            pltpu.SemaphoreType.DMA,
            pltpu.SemaphoreType.DMA,
        ],
    )
    def bag(table, idxf, wgtf, out, idxv, wgtv, r0, r1, accv, s0, s1):
        rowb = (r0, r1)
        semb = (s0, s1)
        wid = lax.axis_index("s") * num_cores + lax.axis_index("c")
        row0 = wid * rows_per_w
        pltpu.sync_copy(idxf.at[pl.ds(row0 * TAPS, rows_per_w * TAPS)], idxv)
        pltpu.sync_copy(wgtf.at[pl.ds(row0 * TAPS, rows_per_w * TAPS)], wgtv)

        def start(g, b):
            idx_slice = idxv.at[pl.ds(g * CHUNK * TAPS, CHUNK * TAPS)]
            pltpu.async_copy(table.at[idx_slice], rowb[b], semb[b])

        def wait(g, b):
            idx_slice = idxv.at[pl.ds(g * CHUNK * TAPS, CHUNK * TAPS)]
            pltpu.make_async_copy(table.at[idx_slice], rowb[b], semb[b]).wait()

        start(0, 0)
        start(1, 1)

        def outer(i, carry):
            for b in (0, 1):
                g = i * 2 + b
                wait(g, b)

                def row_body(r, c2, _rows=rowb[b], _g=g):
                    w16 = wgtv[pl.ds((_g * CHUNK + r) * TAPS, TAPS)]
                    wjs = [_bcast_lane(w16, j) for j in range(TAPS)]
                    for cc in range(cchunks):
                        parts = [None] * 4
                        for j in range(TAPS):
                            v = _rows[r * TAPS + j, pl.ds(cc * LANES, LANES)]
                            k = j % 4
                            parts[k] = (
                                wjs[j] * v
                                if parts[k] is None
                                else parts[k] + wjs[j] * v
                            )
                        accv[r, pl.ds(cc * LANES, LANES)] = (
                            parts[0] + parts[1]
                        ) + (parts[2] + parts[3])
                    return c2

                lax.fori_loop(0, CHUNK, row_body, 0)
                pltpu.sync_copy(accv, out.at[pl.ds(row0 + g * CHUNK, CHUNK)])
                g2 = jnp.minimum(g + 2, n_chunks - 1)
                start(g2, b)
            return carry

        lax.fori_loop(0, n_chunks // 2, outer, 0)
        for b in (0, 1):
            wait(0, b)

    return bag


def kernel(input, rois):
    N, C, H, W = input.shape
    n = rois.shape[0]
    n_bins = OUT_H * OUT_W

    info = plsc.get_sparse_core_info()
    n_workers = info.num_cores * info.num_subcores
    n_pad = n
    while (n_pad * n_bins) % (n_workers * 8):
        n_pad += 8
    n_rows_pad = n_pad * n_bins

    HWC = H * W
    hw_blk = 2048
    table = pl.pallas_call(
        _transpose_kernel,
        grid=(N, HWC // hw_blk),
        in_specs=[
            pl.BlockSpec((1, C, hw_blk), lambda b, j: (b, 0, j)),
        ],
        out_specs=pl.BlockSpec(
            (hw_blk, C), lambda b, j: (b * (HWC // hw_blk) + j, 0)
        ),
        out_shape=jax.ShapeDtypeStruct((N * HWC, C), jnp.float32),
    )(input.reshape(N, C, HWC))

    rois_t = rois.T
    if n_pad != n:
        rois_t = jnp.pad(rois_t, ((0, 0), (0, n_pad - n)))
    roi_blk = 128
    ns = n_bins * TAPS
    idx2, wgt2 = pl.pallas_call(
        functools.partial(_roi_index_kernel, (N, H, W, n)),
        grid=(n_pad // roi_blk,),
        in_specs=[pl.BlockSpec((6, roi_blk), lambda i: (0, i))],
        out_specs=[
            pl.BlockSpec((roi_blk, ns), lambda i: (i, 0)),
            pl.BlockSpec((roi_blk, ns), lambda i: (i, 0)),
        ],
        out_shape=[
            jax.ShapeDtypeStruct((n_pad, ns), jnp.int32),
            jax.ShapeDtypeStruct((n_pad, ns), jnp.float32),
        ],
    )(rois_t)
    idxf = idx2.reshape(n_pad * ns)
    wgtf = wgt2.reshape(n_pad * ns)

    bag = _make_sc_bag(n_rows_pad, C, n_workers, info.num_cores)
    out = bag(table, idxf, wgtf)

    roi_oblk = 8 if n % 8 == 0 else 1
    out_t = pl.pallas_call(
        _out_relayout_kernel,
        grid=(n // roi_oblk,),
        in_specs=[pl.BlockSpec((roi_oblk, n_bins, C), lambda i: (i, 0, 0))],
        out_specs=pl.BlockSpec((roi_oblk, C, n_bins), lambda i: (i, 0, 0)),
        out_shape=jax.ShapeDtypeStruct((n, C, n_bins), jnp.float32),
    )(out.reshape(n_pad, n_bins, C))
    return out_t.reshape(n, C, OUT_H, OUT_W)
